# baseline (speedup 1.0000x reference)
import jax
import jax.numpy as jnp
from jax import lax
from jax.experimental import pallas as pl
from jax.experimental.pallas import tpu as pltpu
from jax.experimental.pallas import tpu_sc as plsc

NUM_WORKERS = 32
LANES = 128


def _lookup_body(uid_idx, iid_idx, tu, ti, dz, outT,
                 idxu_v, idxi_v, ub, ib, colu, coli,
                 sem_u0, sem_u1, sem_u2, sem_u3,
                 sem_i0, sem_i1, sem_i2, sem_i3):
    sems_u = (sem_u0, sem_u1, sem_u2, sem_u3)
    sems_i = (sem_i0, sem_i1, sem_i2, sem_i3)
    D = colu.shape[0]
    n = colu.shape[1]
    wid = lax.axis_index("s") * 2 + lax.axis_index("c")
    base = wid * n
    pltpu.sync_copy(uid_idx.at[pl.ds(base, n)], idxu_v)
    pltpu.sync_copy(iid_idx.at[pl.ds(base, n)], idxi_v)

    r0 = lax.iota(jnp.int32, 16)
    r1 = r0 + 16
    SW = 2
    NQ = 4
    AHEAD = 3

    def fire(vu, vi, s, par):
        for k in range(SW):
            slot = par * SW + k
            iu = vu[s * SW + k]
            ii = vi[s * SW + k]
            cu = pl.multiple_of((iu // LANES) * LANES, LANES)
            ci = pl.multiple_of((ii // LANES) * LANES, LANES)
            for t in range(0, D, 8):
                pltpu.async_copy(
                    tu.at[pl.ds(t, 8), pl.ds(cu, LANES)],
                    ub.at[pl.ds(slot * D + t, 8), :], sems_u[par])
                pltpu.async_copy(
                    ti.at[pl.ds(t, 8), pl.ds(ci, LANES)],
                    ib.at[pl.ds(slot * D + t, 8), :], sems_i[par])

    def drain(par):
        pltpu.make_async_copy(
            dz, ub.at[pl.ds(par * SW * D, SW * D), :], sems_u[par]).wait()
        pltpu.make_async_copy(
            dz, ib.at[pl.ds(par * SW * D, SW * D), :], sems_i[par]).wait()

    slot_r0 = [r0 + slot * D for slot in range(NQ * SW)]
    slot_r1 = [r1 + slot * D for slot in range(NQ * SW)]

    def extract(vu, vi, s, par, j0):
        for k in range(SW):
            slot = par * SW + k
            j = j0 + s * SW + k
            lu = jnp.full((16,), vu[s * SW + k] % LANES, jnp.int32)
            li = jnp.full((16,), vi[s * SW + k] % LANES, jnp.int32)
            cj = jnp.full((16,), j, jnp.int32)
            vu0 = plsc.load_gather(ub, [slot_r0[slot], lu])
            vu1 = plsc.load_gather(ub, [slot_r1[slot], lu])
            vi0 = plsc.load_gather(ib, [slot_r0[slot], li])
            vi1 = plsc.load_gather(ib, [slot_r1[slot], li])
            plsc.store_scatter(colu, [r0, cj], vu0)
            plsc.store_scatter(colu, [r1, cj], vu1)
            plsc.store_scatter(coli, [r0, cj], vi0)
            plsc.store_scatter(coli, [r1, cj], vi1)

    n_sub = 16 // SW
    vu0_ = idxu_v[pl.ds(0, 16)]
    vi0_ = idxi_v[pl.ds(0, 16)]
    for s in range(AHEAD):
        fire(vu0_, vi0_, s, s % NQ)

    def wave(w, carry):
        j0 = w * 16
        vu = idxu_v[pl.ds(j0, 16)]
        vi = idxi_v[pl.ds(j0, 16)]
        jn = lax.rem((w + 1) * 16, n)
        vun = idxu_v[pl.ds(jn, 16)]
        vin = idxi_v[pl.ds(jn, 16)]
        for s in range(n_sub):
            t = s + AHEAD
            if t < n_sub:
                fire(vu, vi, t, t % NQ)
            else:
                fire(vun, vin, t - n_sub, t % NQ)
            drain(s % NQ)
            extract(vu, vi, s, s % NQ, j0)
        return carry

    lax.fori_loop(0, n // 16, wave, 0)
    for s in range(AHEAD):
        drain(s % NQ)

    pltpu.sync_copy(colu, outT.at[0, :, pl.ds(base, n)])
    pltpu.sync_copy(coli, outT.at[1, :, pl.ds(base, n)])


def kernel(x, uid_table, iid_table):
    B = x.shape[0]
    D = uid_table.shape[1]
    n = B // NUM_WORKERS
    uid_idx = x[:, 0].astype(jnp.int32)
    iid_idx = x[:, 1].astype(jnp.int32)
    tu = uid_table.T
    ti = iid_table.T

    mesh = plsc.VectorSubcoreMesh(core_axis_name="c", subcore_axis_name="s")
    run = pl.kernel(
        _lookup_body,
        mesh=mesh,
        out_type=jax.ShapeDtypeStruct((2, D, B), jnp.float32),
        scratch_types=[
            pltpu.VMEM((n,), jnp.int32),
            pltpu.VMEM((n,), jnp.int32),
            pltpu.VMEM((8 * D, LANES), jnp.float32),
            pltpu.VMEM((8 * D, LANES), jnp.float32),
            pltpu.VMEM((D, n), jnp.float32),
            pltpu.VMEM((D, n), jnp.float32),
            pltpu.SemaphoreType.DMA,
            pltpu.SemaphoreType.DMA,
            pltpu.SemaphoreType.DMA,
            pltpu.SemaphoreType.DMA,
            pltpu.SemaphoreType.DMA,
            pltpu.SemaphoreType.DMA,
            pltpu.SemaphoreType.DMA,
            pltpu.SemaphoreType.DMA,
        ],
        compiler_params=pltpu.CompilerParams(needs_layout_passes=False),
    )
    dz = jnp.zeros((2 * D, LANES), jnp.float32)
    outT = run(uid_idx, iid_idx, tu, ti, dz)
    return outT.transpose(2, 0, 1)


# speedup vs baseline: 1.0043x; 1.0043x over previous
import jax
import jax.numpy as jnp
from jax import lax
from jax.experimental import pallas as pl
from jax.experimental.pallas import tpu as pltpu
from jax.experimental.pallas import tpu_sc as plsc

NUM_WORKERS = 32
LANES = 128


def _lookup_body(uid_idx, iid_idx, tu, ti, dz, outT,
                 idxu_v, idxi_v, ub, ib, colu, coli,
                 sem_u0, sem_u1, sem_u2, sem_u3,
                 sem_i0, sem_i1, sem_i2, sem_i3):
    sems_u = (sem_u0, sem_u1, sem_u2, sem_u3)
    sems_i = (sem_i0, sem_i1, sem_i2, sem_i3)
    D = colu.shape[0]
    n = colu.shape[1]
    wid = lax.axis_index("s") * 2 + lax.axis_index("c")
    base = wid * n
    pltpu.sync_copy(uid_idx.at[pl.ds(base, n)], idxu_v)
    pltpu.sync_copy(iid_idx.at[pl.ds(base, n)], idxi_v)

    r0 = lax.iota(jnp.int32, 16)
    r1 = r0 + 16
    SW = 2
    NQ = 4
    AHEAD = 3

    def fire(vu, vi, s, par):
        for k in range(SW):
            slot = par * SW + k
            iu = vu[s * SW + k]
            ii = vi[s * SW + k]
            cu = pl.multiple_of((iu // LANES) * LANES, LANES)
            ci = pl.multiple_of((ii // LANES) * LANES, LANES)
            pltpu.async_copy(
                tu.at[:, pl.ds(cu, LANES)],
                ub.at[pl.ds(slot * D, D), :], sems_u[par])
            pltpu.async_copy(
                ti.at[:, pl.ds(ci, LANES)],
                ib.at[pl.ds(slot * D, D), :], sems_i[par])

    def drain(par):
        pltpu.make_async_copy(
            dz, ub.at[pl.ds(par * SW * D, SW * D), :], sems_u[par]).wait()
        pltpu.make_async_copy(
            dz, ib.at[pl.ds(par * SW * D, SW * D), :], sems_i[par]).wait()

    slot_r0 = [r0 + slot * D for slot in range(NQ * SW)]
    slot_r1 = [r1 + slot * D for slot in range(NQ * SW)]

    def extract(vu, vi, s, par, j0):
        for k in range(SW):
            slot = par * SW + k
            j = j0 + s * SW + k
            lu = jnp.full((16,), vu[s * SW + k] % LANES, jnp.int32)
            li = jnp.full((16,), vi[s * SW + k] % LANES, jnp.int32)
            cj = jnp.full((16,), j, jnp.int32)
            vu0 = plsc.load_gather(ub, [slot_r0[slot], lu])
            vu1 = plsc.load_gather(ub, [slot_r1[slot], lu])
            vi0 = plsc.load_gather(ib, [slot_r0[slot], li])
            vi1 = plsc.load_gather(ib, [slot_r1[slot], li])
            plsc.store_scatter(colu, [r0, cj], vu0)
            plsc.store_scatter(colu, [r1, cj], vu1)
            plsc.store_scatter(coli, [r0, cj], vi0)
            plsc.store_scatter(coli, [r1, cj], vi1)

    n_sub = 16 // SW
    vu0_ = idxu_v[pl.ds(0, 16)]
    vi0_ = idxi_v[pl.ds(0, 16)]
    for s in range(AHEAD):
        fire(vu0_, vi0_, s, s % NQ)

    def wave(w, carry):
        j0 = w * 16
        vu = idxu_v[pl.ds(j0, 16)]
        vi = idxi_v[pl.ds(j0, 16)]
        jn = lax.rem((w + 1) * 16, n)
        vun = idxu_v[pl.ds(jn, 16)]
        vin = idxi_v[pl.ds(jn, 16)]
        for s in range(n_sub):
            t = s + AHEAD
            if t < n_sub:
                fire(vu, vi, t, t % NQ)
            else:
                fire(vun, vin, t - n_sub, t % NQ)
            drain(s % NQ)
            extract(vu, vi, s, s % NQ, j0)
        return carry

    lax.fori_loop(0, n // 16, wave, 0)
    for s in range(AHEAD):
        drain(s % NQ)

    pltpu.sync_copy(colu, outT.at[0, :, pl.ds(base, n)])
    pltpu.sync_copy(coli, outT.at[1, :, pl.ds(base, n)])


def kernel(x, uid_table, iid_table):
    B = x.shape[0]
    D = uid_table.shape[1]
    n = B // NUM_WORKERS
    uid_idx = x[:, 0].astype(jnp.int32)
    iid_idx = x[:, 1].astype(jnp.int32)
    tu = uid_table.T
    ti = iid_table.T

    mesh = plsc.VectorSubcoreMesh(core_axis_name="c", subcore_axis_name="s")
    run = pl.kernel(
        _lookup_body,
        mesh=mesh,
        out_type=jax.ShapeDtypeStruct((2, D, B), jnp.float32),
        scratch_types=[
            pltpu.VMEM((n,), jnp.int32),
            pltpu.VMEM((n,), jnp.int32),
            pltpu.VMEM((8 * D, LANES), jnp.float32),
            pltpu.VMEM((8 * D, LANES), jnp.float32),
            pltpu.VMEM((D, n), jnp.float32),
            pltpu.VMEM((D, n), jnp.float32),
            pltpu.SemaphoreType.DMA,
            pltpu.SemaphoreType.DMA,
            pltpu.SemaphoreType.DMA,
            pltpu.SemaphoreType.DMA,
            pltpu.SemaphoreType.DMA,
            pltpu.SemaphoreType.DMA,
            pltpu.SemaphoreType.DMA,
            pltpu.SemaphoreType.DMA,
        ],
        compiler_params=pltpu.CompilerParams(needs_layout_passes=False),
    )
    dz = jnp.zeros((2 * D, LANES), jnp.float32)
    outT = run(uid_idx, iid_idx, tu, ti, dz)
    return outT.transpose(2, 0, 1)
